# baseline (speedup 1.0000x reference)
import functools

import jax
import jax.numpy as jnp
from jax import lax
from jax.experimental import pallas as pl
from jax.experimental.pallas import tpu as pltpu
from jax.experimental.pallas import tpu_sc as plsc

N_NODES = 100000
N_EDGES = 6400000
NC = 2
NS = 16
L = 16

NPAD = 100352
SCCHUNK = NPAD // NS
CHUNK = NPAD // (NC * NS)
ROWS = 16
BATCH = 128
BLKE = ROWS * BATCH
NBLKG = N_EDGES // BLKE
MAXB = (NBLKG + NC * NS - 1) // (NC * NS)

_mesh = plsc.VectorSubcoreMesh(core_axis_name="c", subcore_axis_name="s")
_f32 = jnp.float32


def _vec_loop(n, body):
    lax.fori_loop(0, n, lambda i, c: (body(i), 0)[1], 0)


def _zero_fill(buf, n):
    z = jnp.zeros((L,), _f32)
    _vec_loop(n // L, lambda i: buf.__setitem__(pl.ds(i * L, L), z))


def _ew(dst, n, fn, *srcs):
    def body(i):
        sl = pl.ds(i * L, L)
        dst[sl] = fn(*[s[sl] for s in srcs])
    _vec_loop(n // L, body)


def _sweep_loop(wid, ei4, vals_sp, idx, gats, accs, ops, semI, semG, semS):
    NW = NC * NS

    def idx_copies(p, t):
        b = wid + t * NW
        return [(ei4.at[r, b], idx[r][p]) for r in (0, 1)]

    def gat_copies(p):
        return [(vals_sp.at[idx[g][p].at[j]], gats[o][p].at[j])
                for o, (g, _) in enumerate(ops) for j in range(ROWS)]

    def sc_copies(p):
        return [(gats[o][p].at[j], accs[o].at[idx[s][p].at[j]])
                for o, (_, s) in enumerate(ops) for j in range(ROWS)]

    def start(copies, sem, add=False):
        for s_ref, d_ref in copies:
            pltpu.async_copy(s_ref, d_ref, sem, add=add)

    def drain(copies, sem):
        for s_ref, d_ref in copies:
            pltpu.make_async_copy(s_ref, d_ref, sem).wait()

    start(idx_copies(0, 0), semI)

    def body(i2, carry):
        for k in range(2):
            t = 2 * i2 + k
            p, q = k, 1 - k

            @pl.when(wid + t * NW < NBLKG)
            def _():
                drain(idx_copies(p, t), semI)
                gc, sc = gat_copies(p), sc_copies(p)
                start(gc, semG)
                for (gs, gd), (ss, sd) in zip(gc, sc):
                    pltpu.make_async_copy(gs, gd, semG).wait()
                    pltpu.async_copy(ss, sd, semS[p], add=True)

            @pl.when((t >= 1) & (wid + (t - 1) * NW < NBLKG))
            def _():
                drain(sc_copies(q), semS[q])

            @pl.when(wid + (t + 1) * NW < NBLKG)
            def _():
                start(idx_copies(q, t + 1), semI)
        return carry

    lax.fori_loop(0, MAXB // 2, body, 0)

    @pl.when(wid + (MAXB - 1) * NW < NBLKG)
    def _():
        drain(sc_copies((MAXB - 1) % 2), semS[(MAXB - 1) % 2])


def _sweep1_body(x0, ei, s0p, vals_sp, acc_sp, st_a,
                 ix0a, ix0b, ix1a, ix1b, ga, gb, semI, semG, semSa, semSb):
    c = lax.axis_index("c")
    s = lax.axis_index("s")
    base = s * SCCHUNK
    sl = pl.ds(base, SCCHUNK)

    _zero_fill(st_a, SCCHUNK)
    pltpu.sync_copy(st_a, acc_sp.at[sl])
    pltpu.sync_copy(x0.at[sl], st_a)
    _ew(st_a, SCCHUNK, lambda v: v * v * v, st_a)
    pltpu.sync_copy(st_a, vals_sp.at[sl])
    plsc.subcore_barrier()

    _sweep_loop(s * NC + c, ei, vals_sp,
                idx=[[ix0a, ix0b], [ix1a, ix1b]],
                gats=[[ga, gb]], accs=[acc_sp], ops=[(0, 1)],
                semI=semI, semG=semG, semS=[semSa, semSb])

    plsc.subcore_barrier()
    pltpu.sync_copy(acc_sp.at[sl], s0p.at[pl.ds(c * NPAD + base, SCCHUNK)])


_idx_scratch = [pltpu.VMEM((ROWS, BATCH), jnp.int32)] * 4
_sem_scratch = [pltpu.SemaphoreType.DMA] * 4

_sweep1 = functools.partial(
    pl.kernel,
    out_type=jax.ShapeDtypeStruct((NC * NPAD,), _f32),
    mesh=_mesh,
    scratch_types=[
        pltpu.VMEM_SHARED((NPAD,), _f32),
        pltpu.VMEM_SHARED((NPAD,), _f32),
        pltpu.VMEM((SCCHUNK,), _f32),
        *_idx_scratch,
        pltpu.VMEM((ROWS, BATCH), _f32),
        pltpu.VMEM((ROWS, BATCH), _f32),
        *_sem_scratch,
    ],
)(_sweep1_body)


def _sweep2_body(x0, s0p, ei, s1p, t1p, x1_out, vals_sp, accs_sp, acct_sp,
                 st_a, st_b, ix0a, ix0b, ix1a, ix1b, ga, gb, g2a, g2b,
                 semI, semG, semSa, semSb):
    c = lax.axis_index("c")
    s = lax.axis_index("s")
    base = s * SCCHUNK
    sl = pl.ds(base, SCCHUNK)

    _zero_fill(st_a, SCCHUNK)
    pltpu.sync_copy(st_a, accs_sp.at[sl])
    pltpu.sync_copy(st_a, acct_sp.at[sl])

    pltpu.sync_copy(s0p.at[pl.ds(base, SCCHUNK)], st_a)
    pltpu.sync_copy(s0p.at[pl.ds(NPAD + base, SCCHUNK)], st_b)
    _ew(st_a, SCCHUNK, lambda a, b: a + b, st_a, st_b)
    pltpu.sync_copy(x0.at[sl], st_b)
    _ew(st_a, SCCHUNK, lambda a, b: b * b * b * a, st_a, st_b)

    @pl.when(c == 0)
    def _():
        pltpu.sync_copy(st_a, x1_out.at[sl])

    _ew(st_a, SCCHUNK, lambda a: a * a * a, st_a)
    pltpu.sync_copy(st_a, vals_sp.at[sl])
    plsc.subcore_barrier()

    _sweep_loop(s * NC + c, ei, vals_sp,
                idx=[[ix0a, ix0b], [ix1a, ix1b]],
                gats=[[ga, gb], [g2a, g2b]], accs=[accs_sp, acct_sp],
                ops=[(0, 1), (1, 0)],
                semI=semI, semG=semG, semS=[semSa, semSb])

    plsc.subcore_barrier()
    pltpu.sync_copy(accs_sp.at[sl], s1p.at[pl.ds(c * NPAD + base, SCCHUNK)])
    pltpu.sync_copy(acct_sp.at[sl], t1p.at[pl.ds(c * NPAD + base, SCCHUNK)])


_sweep2 = functools.partial(
    pl.kernel,
    out_type=(
        jax.ShapeDtypeStruct((NC * NPAD,), _f32),
        jax.ShapeDtypeStruct((NC * NPAD,), _f32),
        jax.ShapeDtypeStruct((NPAD,), _f32),
    ),
    mesh=_mesh,
    scratch_types=[
        pltpu.VMEM_SHARED((NPAD,), _f32),
        pltpu.VMEM_SHARED((NPAD,), _f32),
        pltpu.VMEM_SHARED((NPAD,), _f32),
        pltpu.VMEM((SCCHUNK,), _f32),
        pltpu.VMEM((SCCHUNK,), _f32),
        *_idx_scratch,
        pltpu.VMEM((ROWS, BATCH), _f32),
        pltpu.VMEM((ROWS, BATCH), _f32),
        pltpu.VMEM((ROWS, BATCH), _f32),
        pltpu.VMEM((ROWS, BATCH), _f32),
        *_sem_scratch,
    ],
)(_sweep2_body)


def _sweep3_body(x0, x1, s1p, t1p, ei, up, g1_out, en_out, vals_sp, acc_sp,
                 st_a, st_b, st_c, ix0a, ix0b, ix1a, ix1b, ga, gb, en_v,
                 semI, semG, semSa, semSb):
    c = lax.axis_index("c")
    s = lax.axis_index("s")
    base = s * SCCHUNK
    sl = pl.ds(base, SCCHUNK)

    _zero_fill(st_a, SCCHUNK)
    pltpu.sync_copy(st_a, acc_sp.at[sl])

    pltpu.sync_copy(s1p.at[pl.ds(base, SCCHUNK)], st_a)
    pltpu.sync_copy(s1p.at[pl.ds(NPAD + base, SCCHUNK)], st_b)
    _ew(st_a, SCCHUNK, lambda a, b: a + b, st_a, st_b)
    pltpu.sync_copy(x1.at[sl], st_b)

    @pl.when(c == 0)
    def _():
        def en_body(i, acc):
            slc = pl.ds(i * L, L)
            v = st_b[slc]
            return acc + v * v * v * st_a[slc]
        en = lax.fori_loop(0, SCCHUNK // L, en_body, jnp.zeros((L,), _f32))
        en_v[...] = en
        pltpu.sync_copy(en_v, en_out.at[pl.ds(s * L, L)])

    pltpu.sync_copy(t1p.at[pl.ds(base, SCCHUNK)], st_c)
    _ew(st_a, SCCHUNK, lambda a, x: a + x, st_a, st_c)
    pltpu.sync_copy(t1p.at[pl.ds(NPAD + base, SCCHUNK)], st_c)
    _ew(st_a, SCCHUNK, lambda a, x: a + x, st_a, st_c)
    _ew(st_a, SCCHUNK, lambda a, b: 3.0 * b * b * a, st_a, st_b)

    @pl.when(c == 0)
    def _():
        pltpu.sync_copy(st_a, g1_out.at[sl])

    pltpu.sync_copy(x0.at[sl], st_c)
    _ew(st_a, SCCHUNK, lambda a, x: a * x * x * x, st_a, st_c)
    pltpu.sync_copy(st_a, vals_sp.at[sl])
    plsc.subcore_barrier()

    _sweep_loop(s * NC + c, ei, vals_sp,
                idx=[[ix0a, ix0b], [ix1a, ix1b]],
                gats=[[ga, gb]], accs=[acc_sp], ops=[(1, 0)],
                semI=semI, semG=semG, semS=[semSa, semSb])

    plsc.subcore_barrier()
    pltpu.sync_copy(acc_sp.at[sl], up.at[pl.ds(c * NPAD + base, SCCHUNK)])


_sweep3 = functools.partial(
    pl.kernel,
    out_type=(
        jax.ShapeDtypeStruct((NC * NPAD,), _f32),
        jax.ShapeDtypeStruct((NPAD,), _f32),
        jax.ShapeDtypeStruct((NS * L,), _f32),
    ),
    mesh=_mesh,
    scratch_types=[
        pltpu.VMEM_SHARED((NPAD,), _f32),
        pltpu.VMEM_SHARED((NPAD,), _f32),
        pltpu.VMEM((SCCHUNK,), _f32),
        pltpu.VMEM((SCCHUNK,), _f32),
        pltpu.VMEM((SCCHUNK,), _f32),
        *_idx_scratch,
        pltpu.VMEM((ROWS, BATCH), _f32),
        pltpu.VMEM((ROWS, BATCH), _f32),
        pltpu.VMEM((L,), _f32),
        *_sem_scratch,
    ],
)(_sweep3_body)


def _final_body(x0, g1, s0p, up, forces, st_a, st_b, st_c):
    c = lax.axis_index("c")
    s = lax.axis_index("s")
    wid = s * NC + c
    base = wid * CHUNK
    sl = pl.ds(base, CHUNK)

    pltpu.sync_copy(up.at[pl.ds(base, CHUNK)], st_a)
    pltpu.sync_copy(up.at[pl.ds(NPAD + base, CHUNK)], st_b)
    _ew(st_a, CHUNK, lambda a, b: a + b, st_a, st_b)
    pltpu.sync_copy(s0p.at[pl.ds(base, CHUNK)], st_b)
    pltpu.sync_copy(s0p.at[pl.ds(NPAD + base, CHUNK)], st_c)
    _ew(st_b, CHUNK, lambda a, b: a + b, st_b, st_c)
    pltpu.sync_copy(g1.at[sl], st_c)
    _ew(st_a, CHUNK, lambda a, g, s0: a + g * s0, st_a, st_c, st_b)
    pltpu.sync_copy(x0.at[sl], st_b)
    _ew(st_a, CHUNK, lambda a, x: 3.0 * x * x * a, st_a, st_b)
    pltpu.sync_copy(st_a, forces.at[sl])


_final = functools.partial(
    pl.kernel,
    out_type=jax.ShapeDtypeStruct((NPAD,), _f32),
    mesh=_mesh,
    scratch_types=[
        pltpu.VMEM((CHUNK,), _f32),
        pltpu.VMEM((CHUNK,), _f32),
        pltpu.VMEM((CHUNK,), _f32),
    ],
)(_final_body)


def kernel(atomic_numbers, edge_index):
    x0 = jnp.pad(atomic_numbers, (0, NPAD - N_NODES))
    ei = edge_index.reshape(2, NBLKG, ROWS, BATCH)
    s0p = _sweep1(x0, ei)
    s1p, t1p, x1 = _sweep2(x0, s0p, ei)
    up, g1, en = _sweep3(x0, x1, s1p, t1p, ei)
    forces = _final(x0, g1, s0p, up)
    energy = jnp.sum(en).reshape(1)
    return (energy, forces[:N_NODES])


# speedup vs baseline: 1.1915x; 1.1915x over previous
import functools

import jax
import jax.numpy as jnp
from jax import lax
from jax.experimental import pallas as pl
from jax.experimental.pallas import tpu as pltpu
from jax.experimental.pallas import tpu_sc as plsc

N_NODES = 100000
N_EDGES = 6400000
NC = 2
NS = 16
L = 16

NPAD = 100352
SCCHUNK = NPAD // NS
CHUNK = NPAD // (NC * NS)
ROWS = 16
BATCH = 128
BLKE = ROWS * BATCH
NBLKG = N_EDGES // BLKE
MAXB = (NBLKG + NC * NS - 1) // (NC * NS)

_mesh = plsc.VectorSubcoreMesh(core_axis_name="c", subcore_axis_name="s")
_f32 = jnp.float32


def _vec_loop(n, body):
    lax.fori_loop(0, n, lambda i, c: (body(i), 0)[1], 0)


def _zero_fill(buf, n):
    z = jnp.zeros((L,), _f32)
    _vec_loop(n // L, lambda i: buf.__setitem__(pl.ds(i * L, L), z))


def _ew(dst, n, fn, *srcs):
    def body(i):
        sl = pl.ds(i * L, L)
        dst[sl] = fn(*[s[sl] for s in srcs])
    _vec_loop(n // L, body)


def _sweep_loop(wid, ei4, vals_sp, idx, gats, accs, ops, semI, semG, semS,
                dummy):
    NW = NC * NS

    def idx_copies(p, t):
        b = wid + t * NW
        return [(ei4.at[r, b], idx[r][p]) for r in (0, 1)]

    def gat_copies(p):
        return [(vals_sp.at[idx[g][p].at[j]], gats[o][p].at[j])
                for o, (g, _) in enumerate(ops) for j in range(ROWS)]

    def sc_copies(p):
        return [(gats[o][p].at[j], accs[o].at[idx[s][p].at[j]])
                for o, (_, s) in enumerate(ops) for j in range(ROWS)]

    def start(copies, sem, add=False):
        for s_ref, d_ref in copies:
            pltpu.async_copy(s_ref, d_ref, sem, add=add)

    def drain(copies, sem):
        for s_ref, d_ref in copies:
            pltpu.make_async_copy(s_ref, d_ref, sem).wait()

    def drain_blk(p, sem):
        for o in range(len(ops)):
            pltpu.make_async_copy(dummy, gats[o][p], sem).wait()

    start(idx_copies(0, 0), semI)

    def body(i2, carry):
        for k in range(2):
            t = 2 * i2 + k
            p, q = k, 1 - k

            @pl.when(wid + t * NW < NBLKG)
            def _():
                drain(idx_copies(p, t), semI)
                start(gat_copies(p), semG)
                drain_blk(p, semG)
                start(sc_copies(p), semS[p], add=True)

            @pl.when((t >= 1) & (wid + (t - 1) * NW < NBLKG))
            def _():
                drain_blk(q, semS[q])

            @pl.when(wid + (t + 1) * NW < NBLKG)
            def _():
                start(idx_copies(q, t + 1), semI)
        return carry

    lax.fori_loop(0, MAXB // 2, body, 0)

    @pl.when(wid + (MAXB - 1) * NW < NBLKG)
    def _():
        drain_blk((MAXB - 1) % 2, semS[(MAXB - 1) % 2])


def _sweep1_body(x0, ei, s0p, dum, vals_sp, acc_sp, st_a,
                 ix0a, ix0b, ix1a, ix1b, ga, gb, semI, semG, semSa, semSb):
    c = lax.axis_index("c")
    s = lax.axis_index("s")
    base = s * SCCHUNK
    sl = pl.ds(base, SCCHUNK)

    _zero_fill(st_a, SCCHUNK)
    pltpu.sync_copy(st_a, acc_sp.at[sl])
    pltpu.sync_copy(x0.at[sl], st_a)
    _ew(st_a, SCCHUNK, lambda v: v * v * v, st_a)
    pltpu.sync_copy(st_a, vals_sp.at[sl])
    plsc.subcore_barrier()

    _sweep_loop(s * NC + c, ei, vals_sp,
                idx=[[ix0a, ix0b], [ix1a, ix1b]],
                gats=[[ga, gb]], accs=[acc_sp], ops=[(0, 1)],
                semI=semI, semG=semG, semS=[semSa, semSb], dummy=dum)

    plsc.subcore_barrier()
    pltpu.sync_copy(acc_sp.at[sl], s0p.at[pl.ds(c * NPAD + base, SCCHUNK)])


_idx_scratch = [pltpu.VMEM((ROWS, BATCH), jnp.int32)] * 4
_sem_scratch = [pltpu.SemaphoreType.DMA] * 4

_sweep1 = functools.partial(
    pl.kernel,
    out_type=(jax.ShapeDtypeStruct((NC * NPAD,), _f32),
              jax.ShapeDtypeStruct((ROWS, BATCH), _f32)),
    mesh=_mesh,
    scratch_types=[
        pltpu.VMEM_SHARED((NPAD,), _f32),
        pltpu.VMEM_SHARED((NPAD,), _f32),
        pltpu.VMEM((SCCHUNK,), _f32),
        *_idx_scratch,
        pltpu.VMEM((ROWS, BATCH), _f32),
        pltpu.VMEM((ROWS, BATCH), _f32),
        *_sem_scratch,
    ],
)(_sweep1_body)


def _sweep2_body(x0, s0p, ei, s1p, t1p, x1_out, dum, vals_sp, accs_sp, acct_sp,
                 st_a, st_b, ix0a, ix0b, ix1a, ix1b, ga, gb, g2a, g2b,
                 semI, semG, semSa, semSb):
    c = lax.axis_index("c")
    s = lax.axis_index("s")
    base = s * SCCHUNK
    sl = pl.ds(base, SCCHUNK)

    _zero_fill(st_a, SCCHUNK)
    pltpu.sync_copy(st_a, accs_sp.at[sl])
    pltpu.sync_copy(st_a, acct_sp.at[sl])

    pltpu.sync_copy(s0p.at[pl.ds(base, SCCHUNK)], st_a)
    pltpu.sync_copy(s0p.at[pl.ds(NPAD + base, SCCHUNK)], st_b)
    _ew(st_a, SCCHUNK, lambda a, b: a + b, st_a, st_b)
    pltpu.sync_copy(x0.at[sl], st_b)
    _ew(st_a, SCCHUNK, lambda a, b: b * b * b * a, st_a, st_b)

    @pl.when(c == 0)
    def _():
        pltpu.sync_copy(st_a, x1_out.at[sl])

    _ew(st_a, SCCHUNK, lambda a: a * a * a, st_a)
    pltpu.sync_copy(st_a, vals_sp.at[sl])
    plsc.subcore_barrier()

    _sweep_loop(s * NC + c, ei, vals_sp,
                idx=[[ix0a, ix0b], [ix1a, ix1b]],
                gats=[[ga, gb], [g2a, g2b]], accs=[accs_sp, acct_sp],
                ops=[(0, 1), (1, 0)],
                semI=semI, semG=semG, semS=[semSa, semSb], dummy=dum)

    plsc.subcore_barrier()
    pltpu.sync_copy(accs_sp.at[sl], s1p.at[pl.ds(c * NPAD + base, SCCHUNK)])
    pltpu.sync_copy(acct_sp.at[sl], t1p.at[pl.ds(c * NPAD + base, SCCHUNK)])


_sweep2 = functools.partial(
    pl.kernel,
    out_type=(
        jax.ShapeDtypeStruct((NC * NPAD,), _f32),
        jax.ShapeDtypeStruct((NC * NPAD,), _f32),
        jax.ShapeDtypeStruct((NPAD,), _f32),
        jax.ShapeDtypeStruct((ROWS, BATCH), _f32),
    ),
    mesh=_mesh,
    scratch_types=[
        pltpu.VMEM_SHARED((NPAD,), _f32),
        pltpu.VMEM_SHARED((NPAD,), _f32),
        pltpu.VMEM_SHARED((NPAD,), _f32),
        pltpu.VMEM((SCCHUNK,), _f32),
        pltpu.VMEM((SCCHUNK,), _f32),
        *_idx_scratch,
        pltpu.VMEM((ROWS, BATCH), _f32),
        pltpu.VMEM((ROWS, BATCH), _f32),
        pltpu.VMEM((ROWS, BATCH), _f32),
        pltpu.VMEM((ROWS, BATCH), _f32),
        *_sem_scratch,
    ],
)(_sweep2_body)


def _sweep3_body(x0, x1, s1p, t1p, ei, up, g1_out, en_out, dum, vals_sp, acc_sp,
                 st_a, st_b, st_c, ix0a, ix0b, ix1a, ix1b, ga, gb, en_v,
                 semI, semG, semSa, semSb):
    c = lax.axis_index("c")
    s = lax.axis_index("s")
    base = s * SCCHUNK
    sl = pl.ds(base, SCCHUNK)

    _zero_fill(st_a, SCCHUNK)
    pltpu.sync_copy(st_a, acc_sp.at[sl])

    pltpu.sync_copy(s1p.at[pl.ds(base, SCCHUNK)], st_a)
    pltpu.sync_copy(s1p.at[pl.ds(NPAD + base, SCCHUNK)], st_b)
    _ew(st_a, SCCHUNK, lambda a, b: a + b, st_a, st_b)
    pltpu.sync_copy(x1.at[sl], st_b)

    @pl.when(c == 0)
    def _():
        def en_body(i, acc):
            slc = pl.ds(i * L, L)
            v = st_b[slc]
            return acc + v * v * v * st_a[slc]
        en = lax.fori_loop(0, SCCHUNK // L, en_body, jnp.zeros((L,), _f32))
        en_v[...] = en
        pltpu.sync_copy(en_v, en_out.at[pl.ds(s * L, L)])

    pltpu.sync_copy(t1p.at[pl.ds(base, SCCHUNK)], st_c)
    _ew(st_a, SCCHUNK, lambda a, x: a + x, st_a, st_c)
    pltpu.sync_copy(t1p.at[pl.ds(NPAD + base, SCCHUNK)], st_c)
    _ew(st_a, SCCHUNK, lambda a, x: a + x, st_a, st_c)
    _ew(st_a, SCCHUNK, lambda a, b: 3.0 * b * b * a, st_a, st_b)

    @pl.when(c == 0)
    def _():
        pltpu.sync_copy(st_a, g1_out.at[sl])

    pltpu.sync_copy(x0.at[sl], st_c)
    _ew(st_a, SCCHUNK, lambda a, x: a * x * x * x, st_a, st_c)
    pltpu.sync_copy(st_a, vals_sp.at[sl])
    plsc.subcore_barrier()

    _sweep_loop(s * NC + c, ei, vals_sp,
                idx=[[ix0a, ix0b], [ix1a, ix1b]],
                gats=[[ga, gb]], accs=[acc_sp], ops=[(1, 0)],
                semI=semI, semG=semG, semS=[semSa, semSb], dummy=dum)

    plsc.subcore_barrier()
    pltpu.sync_copy(acc_sp.at[sl], up.at[pl.ds(c * NPAD + base, SCCHUNK)])


_sweep3 = functools.partial(
    pl.kernel,
    out_type=(
        jax.ShapeDtypeStruct((NC * NPAD,), _f32),
        jax.ShapeDtypeStruct((NPAD,), _f32),
        jax.ShapeDtypeStruct((NS * L,), _f32),
        jax.ShapeDtypeStruct((ROWS, BATCH), _f32),
    ),
    mesh=_mesh,
    scratch_types=[
        pltpu.VMEM_SHARED((NPAD,), _f32),
        pltpu.VMEM_SHARED((NPAD,), _f32),
        pltpu.VMEM((SCCHUNK,), _f32),
        pltpu.VMEM((SCCHUNK,), _f32),
        pltpu.VMEM((SCCHUNK,), _f32),
        *_idx_scratch,
        pltpu.VMEM((ROWS, BATCH), _f32),
        pltpu.VMEM((ROWS, BATCH), _f32),
        pltpu.VMEM((L,), _f32),
        *_sem_scratch,
    ],
)(_sweep3_body)


def _final_body(x0, g1, s0p, up, forces, st_a, st_b, st_c):
    c = lax.axis_index("c")
    s = lax.axis_index("s")
    wid = s * NC + c
    base = wid * CHUNK
    sl = pl.ds(base, CHUNK)

    pltpu.sync_copy(up.at[pl.ds(base, CHUNK)], st_a)
    pltpu.sync_copy(up.at[pl.ds(NPAD + base, CHUNK)], st_b)
    _ew(st_a, CHUNK, lambda a, b: a + b, st_a, st_b)
    pltpu.sync_copy(s0p.at[pl.ds(base, CHUNK)], st_b)
    pltpu.sync_copy(s0p.at[pl.ds(NPAD + base, CHUNK)], st_c)
    _ew(st_b, CHUNK, lambda a, b: a + b, st_b, st_c)
    pltpu.sync_copy(g1.at[sl], st_c)
    _ew(st_a, CHUNK, lambda a, g, s0: a + g * s0, st_a, st_c, st_b)
    pltpu.sync_copy(x0.at[sl], st_b)
    _ew(st_a, CHUNK, lambda a, x: 3.0 * x * x * a, st_a, st_b)
    pltpu.sync_copy(st_a, forces.at[sl])


_final = functools.partial(
    pl.kernel,
    out_type=jax.ShapeDtypeStruct((NPAD,), _f32),
    mesh=_mesh,
    scratch_types=[
        pltpu.VMEM((CHUNK,), _f32),
        pltpu.VMEM((CHUNK,), _f32),
        pltpu.VMEM((CHUNK,), _f32),
    ],
)(_final_body)


def kernel(atomic_numbers, edge_index):
    x0 = jnp.pad(atomic_numbers, (0, NPAD - N_NODES))
    ei = edge_index.reshape(2, NBLKG, ROWS, BATCH)
    s0p, _ = _sweep1(x0, ei)
    s1p, t1p, x1, _ = _sweep2(x0, s0p, ei)
    up, g1, en, _ = _sweep3(x0, x1, s1p, t1p, ei)
    forces = _final(x0, g1, s0p, up)
    energy = jnp.sum(en).reshape(1)
    return (energy, forces[:N_NODES])


# speedup vs baseline: 1.2890x; 1.0818x over previous
import functools

import jax
import jax.numpy as jnp
from jax import lax
from jax.experimental import pallas as pl
from jax.experimental.pallas import tpu as pltpu
from jax.experimental.pallas import tpu_sc as plsc

N_NODES = 100000
N_EDGES = 6400000
NC = 2
NS = 16
L = 16

NPAD = 100352
SCCHUNK = NPAD // NS
CHUNK = NPAD // (NC * NS)
ROWS = 16
BATCH = 128
BLKE = ROWS * BATCH
NBLKG = N_EDGES // BLKE
MAXB = (NBLKG + NC * NS - 1) // (NC * NS)

_mesh = plsc.VectorSubcoreMesh(core_axis_name="c", subcore_axis_name="s")
_f32 = jnp.float32


def _vec_loop(n, body):
    lax.fori_loop(0, n, lambda i, c: (body(i), 0)[1], 0)


def _zero_fill(buf, n):
    z = jnp.zeros((L,), _f32)
    _vec_loop(n // L, lambda i: buf.__setitem__(pl.ds(i * L, L), z))


def _ew(dst, n, fn, *srcs):
    def body(i):
        sl = pl.ds(i * L, L)
        dst[sl] = fn(*[s[sl] for s in srcs])
    _vec_loop(n // L, body)


def _sweep_loop(wid, ei4, vals_sp, idx, gats, accs, ops, semI, semG, semS,
                dummy):
    NW = NC * NS

    def idx_copies(p, t):
        b = wid + t * NW
        return [(ei4.at[r, b], idx[r][p]) for r in (0, 1)]

    def gat_copies(p):
        return [(vals_sp.at[idx[g][p].at[j]], gats[o][p].at[j])
                for o, (g, _) in enumerate(ops) for j in range(ROWS)]

    def sc_copies(p):
        return [(gats[o][p].at[j], accs[o].at[idx[s][p].at[j]])
                for o, (_, s) in enumerate(ops) for j in range(ROWS)]

    def start(copies, sem, add=False):
        for s_ref, d_ref in copies:
            pltpu.async_copy(s_ref, d_ref, sem, add=add)

    def drain(copies, sem):
        for s_ref, d_ref in copies:
            pltpu.make_async_copy(s_ref, d_ref, sem).wait()

    def drain_blk(sl, sem):
        for o in range(len(ops)):
            pltpu.make_async_copy(dummy, gats[o][sl], sem).wait()

    def blk_ok(t):
        return wid + t * NW < NBLKG

    start(idx_copies(0, 0), semI)

    def body(i2, carry):
        for k in range(4):
            t = 4 * i2 + k

            @pl.when((t >= 3) & blk_ok(t - 3))
            def _():
                drain_blk((k - 3) % 4, semS[(k - 3) % 2])

            @pl.when(blk_ok(t))
            def _():
                drain(idx_copies(k, t), semI)
                start(gat_copies(k), semG[k % 2])

            @pl.when((t >= 1) & blk_ok(t - 1))
            def _():
                drain_blk((k - 1) % 4, semG[(k - 1) % 2])
                start(sc_copies((k - 1) % 4), semS[(k - 1) % 2], add=True)

            @pl.when(blk_ok(t + 1))
            def _():
                start(idx_copies((k + 1) % 4, t + 1), semI)
        return carry

    lax.fori_loop(0, -(-(MAXB + 3) // 4), body, 0)


def _sweep1_body(x0, ei, s0p, dum, vals_sp, acc_sp, st_a,
                 idxs, gat, semI, semG, semS):
    c = lax.axis_index("c")
    s = lax.axis_index("s")
    base = s * SCCHUNK
    sl = pl.ds(base, SCCHUNK)

    _zero_fill(st_a, SCCHUNK)
    pltpu.sync_copy(st_a, acc_sp.at[sl])
    pltpu.sync_copy(x0.at[sl], st_a)
    _ew(st_a, SCCHUNK, lambda v: v * v * v, st_a)
    pltpu.sync_copy(st_a, vals_sp.at[sl])
    plsc.subcore_barrier()

    _sweep_loop(s * NC + c, ei, vals_sp,
                idx=idxs, gats=gat, accs=[acc_sp], ops=[(0, 1)],
                semI=semI, semG=semG, semS=semS, dummy=dum)

    plsc.subcore_barrier()
    pltpu.sync_copy(acc_sp.at[sl], s0p.at[pl.ds(c * NPAD + base, SCCHUNK)])


def _idx_slots():
    return [[pltpu.VMEM((ROWS, BATCH), jnp.int32) for _ in range(4)]
            for _ in range(2)]


def _gat_slots(nops):
    return [[pltpu.VMEM((ROWS, BATCH), _f32) for _ in range(4)]
            for _ in range(nops)]


_sem_scratch = [pltpu.SemaphoreType.DMA,
                [pltpu.SemaphoreType.DMA] * 2,
                [pltpu.SemaphoreType.DMA] * 2]

_sweep1 = functools.partial(
    pl.kernel,
    out_type=(jax.ShapeDtypeStruct((NC * NPAD,), _f32),
              jax.ShapeDtypeStruct((ROWS, BATCH), _f32)),
    mesh=_mesh,
    scratch_types=[
        pltpu.VMEM_SHARED((NPAD,), _f32),
        pltpu.VMEM_SHARED((NPAD,), _f32),
        pltpu.VMEM((SCCHUNK,), _f32),
        _idx_slots(),
        _gat_slots(1),
        *_sem_scratch,
    ],
)(_sweep1_body)


def _sweep2_body(x0, s0p, ei, s1p, t1p, x1_out, dum, vals_sp, accs_sp, acct_sp,
                 st_a, st_b, idxs, gat, semI, semG, semS):
    c = lax.axis_index("c")
    s = lax.axis_index("s")
    base = s * SCCHUNK
    sl = pl.ds(base, SCCHUNK)

    _zero_fill(st_a, SCCHUNK)
    pltpu.sync_copy(st_a, accs_sp.at[sl])
    pltpu.sync_copy(st_a, acct_sp.at[sl])

    pltpu.sync_copy(s0p.at[pl.ds(base, SCCHUNK)], st_a)
    pltpu.sync_copy(s0p.at[pl.ds(NPAD + base, SCCHUNK)], st_b)
    _ew(st_a, SCCHUNK, lambda a, b: a + b, st_a, st_b)
    pltpu.sync_copy(x0.at[sl], st_b)
    _ew(st_a, SCCHUNK, lambda a, b: b * b * b * a, st_a, st_b)

    @pl.when(c == 0)
    def _():
        pltpu.sync_copy(st_a, x1_out.at[sl])

    _ew(st_a, SCCHUNK, lambda a: a * a * a, st_a)
    pltpu.sync_copy(st_a, vals_sp.at[sl])
    plsc.subcore_barrier()

    _sweep_loop(s * NC + c, ei, vals_sp,
                idx=idxs, gats=gat, accs=[accs_sp, acct_sp],
                ops=[(0, 1), (1, 0)],
                semI=semI, semG=semG, semS=semS, dummy=dum)

    plsc.subcore_barrier()
    pltpu.sync_copy(accs_sp.at[sl], s1p.at[pl.ds(c * NPAD + base, SCCHUNK)])
    pltpu.sync_copy(acct_sp.at[sl], t1p.at[pl.ds(c * NPAD + base, SCCHUNK)])


_sweep2 = functools.partial(
    pl.kernel,
    out_type=(
        jax.ShapeDtypeStruct((NC * NPAD,), _f32),
        jax.ShapeDtypeStruct((NC * NPAD,), _f32),
        jax.ShapeDtypeStruct((NPAD,), _f32),
        jax.ShapeDtypeStruct((ROWS, BATCH), _f32),
    ),
    mesh=_mesh,
    scratch_types=[
        pltpu.VMEM_SHARED((NPAD,), _f32),
        pltpu.VMEM_SHARED((NPAD,), _f32),
        pltpu.VMEM_SHARED((NPAD,), _f32),
        pltpu.VMEM((SCCHUNK,), _f32),
        pltpu.VMEM((SCCHUNK,), _f32),
        _idx_slots(),
        _gat_slots(2),
        *_sem_scratch,
    ],
)(_sweep2_body)


def _sweep3_body(x0, x1, s1p, t1p, ei, up, g1_out, en_out, dum, vals_sp, acc_sp,
                 st_a, st_b, st_c, idxs, gat, en_v, semI, semG, semS):
    c = lax.axis_index("c")
    s = lax.axis_index("s")
    base = s * SCCHUNK
    sl = pl.ds(base, SCCHUNK)

    _zero_fill(st_a, SCCHUNK)
    pltpu.sync_copy(st_a, acc_sp.at[sl])

    pltpu.sync_copy(s1p.at[pl.ds(base, SCCHUNK)], st_a)
    pltpu.sync_copy(s1p.at[pl.ds(NPAD + base, SCCHUNK)], st_b)
    _ew(st_a, SCCHUNK, lambda a, b: a + b, st_a, st_b)
    pltpu.sync_copy(x1.at[sl], st_b)

    @pl.when(c == 0)
    def _():
        def en_body(i, acc):
            slc = pl.ds(i * L, L)
            v = st_b[slc]
            return acc + v * v * v * st_a[slc]
        en = lax.fori_loop(0, SCCHUNK // L, en_body, jnp.zeros((L,), _f32))
        en_v[...] = en
        pltpu.sync_copy(en_v, en_out.at[pl.ds(s * L, L)])

    pltpu.sync_copy(t1p.at[pl.ds(base, SCCHUNK)], st_c)
    _ew(st_a, SCCHUNK, lambda a, x: a + x, st_a, st_c)
    pltpu.sync_copy(t1p.at[pl.ds(NPAD + base, SCCHUNK)], st_c)
    _ew(st_a, SCCHUNK, lambda a, x: a + x, st_a, st_c)
    _ew(st_a, SCCHUNK, lambda a, b: 3.0 * b * b * a, st_a, st_b)

    @pl.when(c == 0)
    def _():
        pltpu.sync_copy(st_a, g1_out.at[sl])

    pltpu.sync_copy(x0.at[sl], st_c)
    _ew(st_a, SCCHUNK, lambda a, x: a * x * x * x, st_a, st_c)
    pltpu.sync_copy(st_a, vals_sp.at[sl])
    plsc.subcore_barrier()

    _sweep_loop(s * NC + c, ei, vals_sp,
                idx=idxs, gats=gat, accs=[acc_sp], ops=[(1, 0)],
                semI=semI, semG=semG, semS=semS, dummy=dum)

    plsc.subcore_barrier()
    pltpu.sync_copy(acc_sp.at[sl], up.at[pl.ds(c * NPAD + base, SCCHUNK)])


_sweep3 = functools.partial(
    pl.kernel,
    out_type=(
        jax.ShapeDtypeStruct((NC * NPAD,), _f32),
        jax.ShapeDtypeStruct((NPAD,), _f32),
        jax.ShapeDtypeStruct((NS * L,), _f32),
        jax.ShapeDtypeStruct((ROWS, BATCH), _f32),
    ),
    mesh=_mesh,
    scratch_types=[
        pltpu.VMEM_SHARED((NPAD,), _f32),
        pltpu.VMEM_SHARED((NPAD,), _f32),
        pltpu.VMEM((SCCHUNK,), _f32),
        pltpu.VMEM((SCCHUNK,), _f32),
        pltpu.VMEM((SCCHUNK,), _f32),
        _idx_slots(),
        _gat_slots(1),
        pltpu.VMEM((L,), _f32),
        *_sem_scratch,
    ],
)(_sweep3_body)


def _final_body(x0, g1, s0p, up, forces, st_a, st_b, st_c):
    c = lax.axis_index("c")
    s = lax.axis_index("s")
    wid = s * NC + c
    base = wid * CHUNK
    sl = pl.ds(base, CHUNK)

    pltpu.sync_copy(up.at[pl.ds(base, CHUNK)], st_a)
    pltpu.sync_copy(up.at[pl.ds(NPAD + base, CHUNK)], st_b)
    _ew(st_a, CHUNK, lambda a, b: a + b, st_a, st_b)
    pltpu.sync_copy(s0p.at[pl.ds(base, CHUNK)], st_b)
    pltpu.sync_copy(s0p.at[pl.ds(NPAD + base, CHUNK)], st_c)
    _ew(st_b, CHUNK, lambda a, b: a + b, st_b, st_c)
    pltpu.sync_copy(g1.at[sl], st_c)
    _ew(st_a, CHUNK, lambda a, g, s0: a + g * s0, st_a, st_c, st_b)
    pltpu.sync_copy(x0.at[sl], st_b)
    _ew(st_a, CHUNK, lambda a, x: 3.0 * x * x * a, st_a, st_b)
    pltpu.sync_copy(st_a, forces.at[sl])


_final = functools.partial(
    pl.kernel,
    out_type=jax.ShapeDtypeStruct((NPAD,), _f32),
    mesh=_mesh,
    scratch_types=[
        pltpu.VMEM((CHUNK,), _f32),
        pltpu.VMEM((CHUNK,), _f32),
        pltpu.VMEM((CHUNK,), _f32),
    ],
)(_final_body)


def kernel(atomic_numbers, edge_index):
    x0 = jnp.pad(atomic_numbers, (0, NPAD - N_NODES))
    ei = edge_index.reshape(2, NBLKG, ROWS, BATCH)
    s0p, _ = _sweep1(x0, ei)
    s1p, t1p, x1, _ = _sweep2(x0, s0p, ei)
    up, g1, en, _ = _sweep3(x0, x1, s1p, t1p, ei)
    forces = _final(x0, g1, s0p, up)
    energy = jnp.sum(en).reshape(1)
    return (energy, forces[:N_NODES])


# speedup vs baseline: 1.3902x; 1.0785x over previous
import functools

import jax
import jax.numpy as jnp
from jax import lax
from jax.experimental import pallas as pl
from jax.experimental.pallas import tpu as pltpu
from jax.experimental.pallas import tpu_sc as plsc

N_NODES = 100000
N_EDGES = 6400000
NC = 2
NS = 16
L = 16

NPAD = 100352
SCCHUNK = NPAD // NS
CHUNK = NPAD // (NC * NS)
ROWS = 16
BATCH = 128
BLKE = ROWS * BATCH
NBLKG = N_EDGES // BLKE
MAXB = (NBLKG + NC * NS - 1) // (NC * NS)

_mesh = plsc.VectorSubcoreMesh(core_axis_name="c", subcore_axis_name="s")
_f32 = jnp.float32


def _vec_loop(n, body):
    lax.fori_loop(0, n, lambda i, c: (body(i), 0)[1], 0)


def _zero_fill(buf, n):
    z = jnp.zeros((L,), _f32)
    _vec_loop(n // L, lambda i: buf.__setitem__(pl.ds(i * L, L), z))


def _ew(dst, n, fn, *srcs):
    def body(i):
        sl = pl.ds(i * L, L)
        dst[sl] = fn(*[s[sl] for s in srcs])
    _vec_loop(n // L, body)


def _sweep_loop(wid, ei4, vals_sp, idx, gats, accs, ops, semI, semG, semS,
                dummy):
    NW = NC * NS

    def idx_copies(p, t):
        b = wid + t * NW
        return [(ei4.at[r, b], idx[r][p]) for r in (0, 1)]

    def gat_copies(p):
        return [(vals_sp.at[idx[g][p].at[j]], gats[o][p].at[j])
                for o, (g, _) in enumerate(ops) for j in range(ROWS)]

    def sc_copies(p):
        return [(gats[o][p].at[j], accs[o].at[idx[s][p].at[j]])
                for o, (_, s) in enumerate(ops) for j in range(ROWS)]

    def start(copies, sem, add=False):
        for s_ref, d_ref in copies:
            pltpu.async_copy(s_ref, d_ref, sem, add=add)

    def drain(copies, sem):
        for s_ref, d_ref in copies:
            pltpu.make_async_copy(s_ref, d_ref, sem).wait()

    def drain_blk(sl, sem):
        for o in range(len(ops)):
            pltpu.make_async_copy(dummy, gats[o][sl], sem).wait()

    def blk_ok(t):
        return wid + t * NW < NBLKG

    start(idx_copies(0, 0), semI)

    def body(i2, carry):
        for k in range(4):
            t = 4 * i2 + k

            @pl.when((t >= 3) & blk_ok(t - 3))
            def _():
                drain_blk((k - 3) % 4, semS[(k - 3) % 2])

            @pl.when(blk_ok(t))
            def _():
                drain(idx_copies(k, t), semI)
                start(gat_copies(k), semG[k % 2])

            @pl.when((t >= 1) & blk_ok(t - 1))
            def _():
                drain_blk((k - 1) % 4, semG[(k - 1) % 2])
                start(sc_copies((k - 1) % 4), semS[(k - 1) % 2], add=True)

            @pl.when(blk_ok(t + 1))
            def _():
                start(idx_copies((k + 1) % 4, t + 1), semI)
        return carry

    lax.fori_loop(0, -(-(MAXB + 3) // 4), body, 0)


def _sweep1_body(x0, ei, s0p, dum, vals_sp, acc_sp, st_a,
                 idxs, gat, semI, semG, semS):
    c = lax.axis_index("c")
    s = lax.axis_index("s")
    base = s * SCCHUNK
    sl = pl.ds(base, SCCHUNK)

    _zero_fill(st_a, SCCHUNK)
    pltpu.sync_copy(st_a, acc_sp.at[sl])
    pltpu.sync_copy(x0.at[sl], st_a)
    _ew(st_a, SCCHUNK, lambda v: v * v * v, st_a)
    pltpu.sync_copy(st_a, vals_sp.at[sl])
    plsc.subcore_barrier()

    _sweep_loop(s * NC + c, ei, vals_sp,
                idx=idxs, gats=gat, accs=[acc_sp], ops=[(0, 1)],
                semI=semI, semG=semG, semS=semS, dummy=dum)

    plsc.subcore_barrier()
    pltpu.sync_copy(acc_sp.at[sl], s0p.at[pl.ds(c * NPAD + base, SCCHUNK)])


def _idx_slots():
    return [[pltpu.VMEM((ROWS, BATCH), jnp.int32) for _ in range(4)]
            for _ in range(2)]


def _gat_slots(nops):
    return [[pltpu.VMEM((ROWS, BATCH), _f32) for _ in range(4)]
            for _ in range(nops)]


_sem_scratch = [pltpu.SemaphoreType.DMA,
                [pltpu.SemaphoreType.DMA] * 2,
                [pltpu.SemaphoreType.DMA] * 2]

_sweep1 = functools.partial(
    pl.kernel,
    out_type=(jax.ShapeDtypeStruct((NC * NPAD,), _f32),
              jax.ShapeDtypeStruct((ROWS, BATCH), _f32)),
    mesh=_mesh,
    scratch_types=[
        pltpu.VMEM_SHARED((NPAD,), _f32),
        pltpu.VMEM_SHARED((NPAD,), _f32),
        pltpu.VMEM((SCCHUNK,), _f32),
        _idx_slots(),
        _gat_slots(1),
        *_sem_scratch,
    ],
)(_sweep1_body)


def _sweep2_body(x0, s0p, ei, a1p, x1_out, dum, vals_sp, acc_sp,
                 st_a, st_b, idxs, gat, semI, semG, semS):
    c = lax.axis_index("c")
    s = lax.axis_index("s")
    base = s * SCCHUNK
    sl = pl.ds(base, SCCHUNK)

    _zero_fill(st_a, SCCHUNK)
    pltpu.sync_copy(st_a, acc_sp.at[sl])

    pltpu.sync_copy(s0p.at[pl.ds(base, SCCHUNK)], st_a)
    pltpu.sync_copy(s0p.at[pl.ds(NPAD + base, SCCHUNK)], st_b)
    _ew(st_a, SCCHUNK, lambda a, b: a + b, st_a, st_b)
    pltpu.sync_copy(x0.at[sl], st_b)
    _ew(st_a, SCCHUNK, lambda a, b: b * b * b * a, st_a, st_b)

    @pl.when(c == 0)
    def _():
        pltpu.sync_copy(st_a, x1_out.at[sl])

    _ew(st_a, SCCHUNK, lambda a: a * a * a, st_a)
    pltpu.sync_copy(st_a, vals_sp.at[sl])
    plsc.subcore_barrier()

    _sweep_loop(s * NC + c, ei, vals_sp,
                idx=idxs, gats=gat, accs=[acc_sp], ops=[(0, 1)],
                semI=semI, semG=semG, semS=semS, dummy=dum)
    _sweep_loop(s * NC + c, ei, vals_sp,
                idx=idxs, gats=gat, accs=[acc_sp], ops=[(1, 0)],
                semI=semI, semG=semG, semS=semS, dummy=dum)

    plsc.subcore_barrier()
    pltpu.sync_copy(acc_sp.at[sl], a1p.at[pl.ds(c * NPAD + base, SCCHUNK)])


_sweep2 = functools.partial(
    pl.kernel,
    out_type=(
        jax.ShapeDtypeStruct((NC * NPAD,), _f32),
        jax.ShapeDtypeStruct((NPAD,), _f32),
        jax.ShapeDtypeStruct((ROWS, BATCH), _f32),
    ),
    mesh=_mesh,
    scratch_types=[
        pltpu.VMEM_SHARED((NPAD,), _f32),
        pltpu.VMEM_SHARED((NPAD,), _f32),
        pltpu.VMEM((SCCHUNK,), _f32),
        pltpu.VMEM((SCCHUNK,), _f32),
        _idx_slots(),
        _gat_slots(1),
        *_sem_scratch,
    ],
)(_sweep2_body)


def _sweep3_body(x0, x1, a1p, ei, up, g1_out, en_out, dum, vals_sp, acc_sp,
                 st_a, st_b, st_c, idxs, gat, en_v, semI, semG, semS):
    c = lax.axis_index("c")
    s = lax.axis_index("s")
    base = s * SCCHUNK
    sl = pl.ds(base, SCCHUNK)

    _zero_fill(st_a, SCCHUNK)
    pltpu.sync_copy(st_a, acc_sp.at[sl])

    pltpu.sync_copy(a1p.at[pl.ds(base, SCCHUNK)], st_a)
    pltpu.sync_copy(a1p.at[pl.ds(NPAD + base, SCCHUNK)], st_b)
    _ew(st_a, SCCHUNK, lambda a, b: a + b, st_a, st_b)
    pltpu.sync_copy(x1.at[sl], st_b)

    @pl.when(c == 0)
    def _():
        def en_body(i, acc):
            slc = pl.ds(i * L, L)
            v = st_b[slc]
            return acc + v * v * v * st_a[slc]
        en = lax.fori_loop(0, SCCHUNK // L, en_body, jnp.zeros((L,), _f32))
        en_v[...] = 0.5 * en
        pltpu.sync_copy(en_v, en_out.at[pl.ds(s * L, L)])

    _ew(st_a, SCCHUNK, lambda a, b: 3.0 * b * b * a, st_a, st_b)

    @pl.when(c == 0)
    def _():
        pltpu.sync_copy(st_a, g1_out.at[sl])

    pltpu.sync_copy(x0.at[sl], st_c)
    _ew(st_a, SCCHUNK, lambda a, x: a * x * x * x, st_a, st_c)
    pltpu.sync_copy(st_a, vals_sp.at[sl])
    plsc.subcore_barrier()

    _sweep_loop(s * NC + c, ei, vals_sp,
                idx=idxs, gats=gat, accs=[acc_sp], ops=[(1, 0)],
                semI=semI, semG=semG, semS=semS, dummy=dum)

    plsc.subcore_barrier()
    pltpu.sync_copy(acc_sp.at[sl], up.at[pl.ds(c * NPAD + base, SCCHUNK)])


_sweep3 = functools.partial(
    pl.kernel,
    out_type=(
        jax.ShapeDtypeStruct((NC * NPAD,), _f32),
        jax.ShapeDtypeStruct((NPAD,), _f32),
        jax.ShapeDtypeStruct((NS * L,), _f32),
        jax.ShapeDtypeStruct((ROWS, BATCH), _f32),
    ),
    mesh=_mesh,
    scratch_types=[
        pltpu.VMEM_SHARED((NPAD,), _f32),
        pltpu.VMEM_SHARED((NPAD,), _f32),
        pltpu.VMEM((SCCHUNK,), _f32),
        pltpu.VMEM((SCCHUNK,), _f32),
        pltpu.VMEM((SCCHUNK,), _f32),
        _idx_slots(),
        _gat_slots(1),
        pltpu.VMEM((L,), _f32),
        *_sem_scratch,
    ],
)(_sweep3_body)


def _final_body(x0, g1, s0p, up, forces, st_a, st_b, st_c):
    c = lax.axis_index("c")
    s = lax.axis_index("s")
    wid = s * NC + c
    base = wid * CHUNK
    sl = pl.ds(base, CHUNK)

    pltpu.sync_copy(up.at[pl.ds(base, CHUNK)], st_a)
    pltpu.sync_copy(up.at[pl.ds(NPAD + base, CHUNK)], st_b)
    _ew(st_a, CHUNK, lambda a, b: a + b, st_a, st_b)
    pltpu.sync_copy(s0p.at[pl.ds(base, CHUNK)], st_b)
    pltpu.sync_copy(s0p.at[pl.ds(NPAD + base, CHUNK)], st_c)
    _ew(st_b, CHUNK, lambda a, b: a + b, st_b, st_c)
    pltpu.sync_copy(g1.at[sl], st_c)
    _ew(st_a, CHUNK, lambda a, g, s0: a + g * s0, st_a, st_c, st_b)
    pltpu.sync_copy(x0.at[sl], st_b)
    _ew(st_a, CHUNK, lambda a, x: 3.0 * x * x * a, st_a, st_b)
    pltpu.sync_copy(st_a, forces.at[sl])


_final = functools.partial(
    pl.kernel,
    out_type=jax.ShapeDtypeStruct((NPAD,), _f32),
    mesh=_mesh,
    scratch_types=[
        pltpu.VMEM((CHUNK,), _f32),
        pltpu.VMEM((CHUNK,), _f32),
        pltpu.VMEM((CHUNK,), _f32),
    ],
)(_final_body)


def kernel(atomic_numbers, edge_index):
    x0 = jnp.pad(atomic_numbers, (0, NPAD - N_NODES))
    ei = edge_index.reshape(2, NBLKG, ROWS, BATCH)
    s0p, _ = _sweep1(x0, ei)
    a1p, x1, _ = _sweep2(x0, s0p, ei)
    up, g1, en, _ = _sweep3(x0, x1, a1p, ei)
    forces = _final(x0, g1, s0p, up)
    energy = jnp.sum(en).reshape(1)
    return (energy, forces[:N_NODES])


# speedup vs baseline: 1.4223x; 1.0231x over previous
import functools

import jax
import jax.numpy as jnp
from jax import lax
from jax.experimental import pallas as pl
from jax.experimental.pallas import tpu as pltpu
from jax.experimental.pallas import tpu_sc as plsc

N_NODES = 100000
N_EDGES = 6400000
NC = 2
NS = 16
L = 16

NPAD = 100352
SCCHUNK = NPAD // NS
CHUNK = NPAD // (NC * NS)
ROWS = 16
BATCH = 128
BLKE = ROWS * BATCH
NBLKG = N_EDGES // BLKE
MAXB = (NBLKG + NC * NS - 1) // (NC * NS)

_mesh = plsc.VectorSubcoreMesh(core_axis_name="c", subcore_axis_name="s")
_f32 = jnp.float32


def _vec_loop(n, body):
    lax.fori_loop(0, n, lambda i, c: (body(i), 0)[1], 0)


def _zero_fill(buf, n):
    z = jnp.zeros((L,), _f32)
    _vec_loop(n // L, lambda i: buf.__setitem__(pl.ds(i * L, L), z))


def _ew(dst, n, fn, *srcs):
    def body(i):
        sl = pl.ds(i * L, L)
        dst[sl] = fn(*[s[sl] for s in srcs])
    _vec_loop(n // L, body)


def _sweep_loop(wid, ei4, vals_sp, idx, gats, accs, ops, semI, semG, semS,
                dummy):
    NW = NC * NS

    def idx_copies(p, t):
        b = wid + t * NW
        return [(ei4.at[r, b], idx[r][p]) for r in (0, 1)]

    def gat_copies(p):
        return [(vals_sp.at[idx[g][p].at[j]], gats[o][p].at[j])
                for o, (g, _) in enumerate(ops) for j in range(ROWS)]

    def sc_copies(p):
        return [(gats[o][p].at[j], accs[o].at[idx[s][p].at[j]])
                for o, (_, s) in enumerate(ops) for j in range(ROWS)]

    def start(copies, sem, add=False):
        for s_ref, d_ref in copies:
            pltpu.async_copy(s_ref, d_ref, sem, add=add)

    def drain(copies, sem):
        for s_ref, d_ref in copies:
            pltpu.make_async_copy(s_ref, d_ref, sem).wait()

    def drain_blk(sl, sem):
        for o in range(len(ops)):
            pltpu.make_async_copy(dummy, gats[o][sl], sem).wait()

    def blk_ok(t):
        return wid + t * NW < NBLKG

    start(idx_copies(0, 0), semI)

    def body(i2, carry):
        for k in range(6):
            t = 6 * i2 + k

            @pl.when((t >= 5) & blk_ok(t - 5))
            def _():
                drain_blk((k - 5) % 6, semS[(k - 5) % 3])

            @pl.when(blk_ok(t))
            def _():
                drain(idx_copies(k % 6, t), semI)
                start(gat_copies(k % 6), semG[k % 3])

            @pl.when((t >= 2) & blk_ok(t - 2))
            def _():
                drain_blk((k - 2) % 6, semG[(k - 2) % 3])
                start(sc_copies((k - 2) % 6), semS[(k - 2) % 3], add=True)

            @pl.when(blk_ok(t + 1))
            def _():
                start(idx_copies((k + 1) % 6, t + 1), semI)
        return carry

    lax.fori_loop(0, -(-(MAXB + 5) // 6), body, 0)


def _sweep1_body(x0, ei, s0p, dum, vals_sp, acc_sp, st_a,
                 idxs, gat, semI, semG, semS):
    c = lax.axis_index("c")
    s = lax.axis_index("s")
    base = s * SCCHUNK
    sl = pl.ds(base, SCCHUNK)

    _zero_fill(st_a, SCCHUNK)
    pltpu.sync_copy(st_a, acc_sp.at[sl])
    pltpu.sync_copy(x0.at[sl], st_a)
    _ew(st_a, SCCHUNK, lambda v: v * v * v, st_a)
    pltpu.sync_copy(st_a, vals_sp.at[sl])
    plsc.subcore_barrier()

    _sweep_loop(s * NC + c, ei, vals_sp,
                idx=idxs, gats=gat, accs=[acc_sp], ops=[(0, 1)],
                semI=semI, semG=semG, semS=semS, dummy=dum)

    plsc.subcore_barrier()
    pltpu.sync_copy(acc_sp.at[sl], s0p.at[pl.ds(c * NPAD + base, SCCHUNK)])


def _idx_slots():
    return [[pltpu.VMEM((ROWS, BATCH), jnp.int32) for _ in range(6)]
            for _ in range(2)]


def _gat_slots(nops):
    return [[pltpu.VMEM((ROWS, BATCH), _f32) for _ in range(6)]
            for _ in range(nops)]


_sem_scratch = [pltpu.SemaphoreType.DMA,
                [pltpu.SemaphoreType.DMA] * 3,
                [pltpu.SemaphoreType.DMA] * 3]

_sweep1 = functools.partial(
    pl.kernel,
    out_type=(jax.ShapeDtypeStruct((NC * NPAD,), _f32),
              jax.ShapeDtypeStruct((ROWS, BATCH), _f32)),
    mesh=_mesh,
    scratch_types=[
        pltpu.VMEM_SHARED((NPAD,), _f32),
        pltpu.VMEM_SHARED((NPAD,), _f32),
        pltpu.VMEM((SCCHUNK,), _f32),
        _idx_slots(),
        _gat_slots(1),
        *_sem_scratch,
    ],
)(_sweep1_body)


def _sweep2_body(x0, s0p, ei, a1p, x1_out, dum, vals_sp, acc_sp,
                 st_a, st_b, idxs, gat, semI, semG, semS):
    c = lax.axis_index("c")
    s = lax.axis_index("s")
    base = s * SCCHUNK
    sl = pl.ds(base, SCCHUNK)

    _zero_fill(st_a, SCCHUNK)
    pltpu.sync_copy(st_a, acc_sp.at[sl])

    pltpu.sync_copy(s0p.at[pl.ds(base, SCCHUNK)], st_a)
    pltpu.sync_copy(s0p.at[pl.ds(NPAD + base, SCCHUNK)], st_b)
    _ew(st_a, SCCHUNK, lambda a, b: a + b, st_a, st_b)
    pltpu.sync_copy(x0.at[sl], st_b)
    _ew(st_a, SCCHUNK, lambda a, b: b * b * b * a, st_a, st_b)

    @pl.when(c == 0)
    def _():
        pltpu.sync_copy(st_a, x1_out.at[sl])

    _ew(st_a, SCCHUNK, lambda a: a * a * a, st_a)
    pltpu.sync_copy(st_a, vals_sp.at[sl])
    plsc.subcore_barrier()

    _sweep_loop(s * NC + c, ei, vals_sp,
                idx=idxs, gats=gat, accs=[acc_sp], ops=[(0, 1)],
                semI=semI, semG=semG, semS=semS, dummy=dum)
    _sweep_loop(s * NC + c, ei, vals_sp,
                idx=idxs, gats=gat, accs=[acc_sp], ops=[(1, 0)],
                semI=semI, semG=semG, semS=semS, dummy=dum)

    plsc.subcore_barrier()
    pltpu.sync_copy(acc_sp.at[sl], a1p.at[pl.ds(c * NPAD + base, SCCHUNK)])


_sweep2 = functools.partial(
    pl.kernel,
    out_type=(
        jax.ShapeDtypeStruct((NC * NPAD,), _f32),
        jax.ShapeDtypeStruct((NPAD,), _f32),
        jax.ShapeDtypeStruct((ROWS, BATCH), _f32),
    ),
    mesh=_mesh,
    scratch_types=[
        pltpu.VMEM_SHARED((NPAD,), _f32),
        pltpu.VMEM_SHARED((NPAD,), _f32),
        pltpu.VMEM((SCCHUNK,), _f32),
        pltpu.VMEM((SCCHUNK,), _f32),
        _idx_slots(),
        _gat_slots(1),
        *_sem_scratch,
    ],
)(_sweep2_body)


def _sweep3_body(x0, x1, a1p, ei, up, g1_out, en_out, dum, vals_sp, acc_sp,
                 st_a, st_b, st_c, idxs, gat, en_v, semI, semG, semS):
    c = lax.axis_index("c")
    s = lax.axis_index("s")
    base = s * SCCHUNK
    sl = pl.ds(base, SCCHUNK)

    _zero_fill(st_a, SCCHUNK)
    pltpu.sync_copy(st_a, acc_sp.at[sl])

    pltpu.sync_copy(a1p.at[pl.ds(base, SCCHUNK)], st_a)
    pltpu.sync_copy(a1p.at[pl.ds(NPAD + base, SCCHUNK)], st_b)
    _ew(st_a, SCCHUNK, lambda a, b: a + b, st_a, st_b)
    pltpu.sync_copy(x1.at[sl], st_b)

    @pl.when(c == 0)
    def _():
        def en_body(i, acc):
            slc = pl.ds(i * L, L)
            v = st_b[slc]
            return acc + v * v * v * st_a[slc]
        en = lax.fori_loop(0, SCCHUNK // L, en_body, jnp.zeros((L,), _f32))
        en_v[...] = 0.5 * en
        pltpu.sync_copy(en_v, en_out.at[pl.ds(s * L, L)])

    _ew(st_a, SCCHUNK, lambda a, b: 3.0 * b * b * a, st_a, st_b)

    @pl.when(c == 0)
    def _():
        pltpu.sync_copy(st_a, g1_out.at[sl])

    pltpu.sync_copy(x0.at[sl], st_c)
    _ew(st_a, SCCHUNK, lambda a, x: a * x * x * x, st_a, st_c)
    pltpu.sync_copy(st_a, vals_sp.at[sl])
    plsc.subcore_barrier()

    _sweep_loop(s * NC + c, ei, vals_sp,
                idx=idxs, gats=gat, accs=[acc_sp], ops=[(1, 0)],
                semI=semI, semG=semG, semS=semS, dummy=dum)

    plsc.subcore_barrier()
    pltpu.sync_copy(acc_sp.at[sl], up.at[pl.ds(c * NPAD + base, SCCHUNK)])


_sweep3 = functools.partial(
    pl.kernel,
    out_type=(
        jax.ShapeDtypeStruct((NC * NPAD,), _f32),
        jax.ShapeDtypeStruct((NPAD,), _f32),
        jax.ShapeDtypeStruct((NS * L,), _f32),
        jax.ShapeDtypeStruct((ROWS, BATCH), _f32),
    ),
    mesh=_mesh,
    scratch_types=[
        pltpu.VMEM_SHARED((NPAD,), _f32),
        pltpu.VMEM_SHARED((NPAD,), _f32),
        pltpu.VMEM((SCCHUNK,), _f32),
        pltpu.VMEM((SCCHUNK,), _f32),
        pltpu.VMEM((SCCHUNK,), _f32),
        _idx_slots(),
        _gat_slots(1),
        pltpu.VMEM((L,), _f32),
        *_sem_scratch,
    ],
)(_sweep3_body)


def _final_body(x0, g1, s0p, up, forces, st_a, st_b, st_c):
    c = lax.axis_index("c")
    s = lax.axis_index("s")
    wid = s * NC + c
    base = wid * CHUNK
    sl = pl.ds(base, CHUNK)

    pltpu.sync_copy(up.at[pl.ds(base, CHUNK)], st_a)
    pltpu.sync_copy(up.at[pl.ds(NPAD + base, CHUNK)], st_b)
    _ew(st_a, CHUNK, lambda a, b: a + b, st_a, st_b)
    pltpu.sync_copy(s0p.at[pl.ds(base, CHUNK)], st_b)
    pltpu.sync_copy(s0p.at[pl.ds(NPAD + base, CHUNK)], st_c)
    _ew(st_b, CHUNK, lambda a, b: a + b, st_b, st_c)
    pltpu.sync_copy(g1.at[sl], st_c)
    _ew(st_a, CHUNK, lambda a, g, s0: a + g * s0, st_a, st_c, st_b)
    pltpu.sync_copy(x0.at[sl], st_b)
    _ew(st_a, CHUNK, lambda a, x: 3.0 * x * x * a, st_a, st_b)
    pltpu.sync_copy(st_a, forces.at[sl])


_final = functools.partial(
    pl.kernel,
    out_type=jax.ShapeDtypeStruct((NPAD,), _f32),
    mesh=_mesh,
    scratch_types=[
        pltpu.VMEM((CHUNK,), _f32),
        pltpu.VMEM((CHUNK,), _f32),
        pltpu.VMEM((CHUNK,), _f32),
    ],
)(_final_body)


def kernel(atomic_numbers, edge_index):
    x0 = jnp.pad(atomic_numbers, (0, NPAD - N_NODES))
    ei = edge_index.reshape(2, NBLKG, ROWS, BATCH)
    s0p, _ = _sweep1(x0, ei)
    a1p, x1, _ = _sweep2(x0, s0p, ei)
    up, g1, en, _ = _sweep3(x0, x1, a1p, ei)
    forces = _final(x0, g1, s0p, up)
    energy = jnp.sum(en).reshape(1)
    return (energy, forces[:N_NODES])


# speedup vs baseline: 1.4223x; 1.0000x over previous
import functools

import jax
import jax.numpy as jnp
from jax import lax
from jax.experimental import pallas as pl
from jax.experimental.pallas import tpu as pltpu
from jax.experimental.pallas import tpu_sc as plsc

N_NODES = 100000
N_EDGES = 6400000
NC = 2
NS = 16
L = 16

NPAD = 100352
SCCHUNK = NPAD // NS
CHUNK = NPAD // (NC * NS)
ROWS = 16
BATCH = 128
BLKE = ROWS * BATCH
NBLKG = N_EDGES // BLKE
MAXB = (NBLKG + NC * NS - 1) // (NC * NS)

_mesh = plsc.VectorSubcoreMesh(core_axis_name="c", subcore_axis_name="s",
                               num_cores=NC, num_subcores=NS)
_f32 = jnp.float32


def _vec_loop(n, body):
    lax.fori_loop(0, n, lambda i, c: (body(i), 0)[1], 0)


def _zero_fill(buf, n):
    z = jnp.zeros((L,), _f32)
    _vec_loop(n // L, lambda i: buf.__setitem__(pl.ds(i * L, L), z))


def _ew(dst, n, fn, *srcs):
    def body(i):
        sl = pl.ds(i * L, L)
        dst[sl] = fn(*[s[sl] for s in srcs])
    _vec_loop(n // L, body)


def _sweep_loop(wid, ei4, vals_sp, idx, gats, accs, ops, semI, semG, semS,
                dummy):
    NW = NC * NS

    def idx_copies(p, t):
        b = wid + t * NW
        return [(ei4.at[r, b], idx[r][p]) for r in (0, 1)]

    def gat_copies(p):
        return [(vals_sp.at[idx[g][p].at[j]], gats[o][p].at[j])
                for o, (g, _) in enumerate(ops) for j in range(ROWS)]

    def sc_copies(p):
        return [(gats[o][p].at[j], accs[o].at[idx[s][p].at[j]])
                for o, (_, s) in enumerate(ops) for j in range(ROWS)]

    def start(copies, sem, add=False):
        for s_ref, d_ref in copies:
            pltpu.async_copy(s_ref, d_ref, sem, add=add)

    def drain(copies, sem):
        for s_ref, d_ref in copies:
            pltpu.make_async_copy(s_ref, d_ref, sem).wait()

    def drain_blk(sl, sem):
        for o in range(len(ops)):
            pltpu.make_async_copy(dummy, gats[o][sl], sem).wait()

    def blk_ok(t):
        return wid + t * NW < NBLKG

    start(idx_copies(0, 0), semI)

    def body(i2, carry):
        for k in range(6):
            t = 6 * i2 + k

            @pl.when((t >= 5) & blk_ok(t - 5))
            def _():
                drain_blk((k - 5) % 6, semS[(k - 5) % 3])

            @pl.when(blk_ok(t))
            def _():
                drain(idx_copies(k % 6, t), semI)
                start(gat_copies(k % 6), semG[k % 3])

            @pl.when((t >= 2) & blk_ok(t - 2))
            def _():
                drain_blk((k - 2) % 6, semG[(k - 2) % 3])
                start(sc_copies((k - 2) % 6), semS[(k - 2) % 3], add=True)

            @pl.when(blk_ok(t + 1))
            def _():
                start(idx_copies((k + 1) % 6, t + 1), semI)
        return carry

    lax.fori_loop(0, -(-(MAXB + 5) // 6), body, 0)


def _sweep1_body(x0, ei, s0p, dum, vals_sp, acc_sp, st_a,
                 idxs, gat, semI, semG, semS):
    c = lax.axis_index("c")
    s = lax.axis_index("s")
    base = s * SCCHUNK
    sl = pl.ds(base, SCCHUNK)

    _zero_fill(st_a, SCCHUNK)
    pltpu.sync_copy(st_a, acc_sp.at[sl])
    pltpu.sync_copy(x0.at[sl], st_a)
    _ew(st_a, SCCHUNK, lambda v: v * v * v, st_a)
    pltpu.sync_copy(st_a, vals_sp.at[sl])
    plsc.subcore_barrier()

    _sweep_loop(s * NC + c, ei, vals_sp,
                idx=idxs, gats=gat, accs=[acc_sp], ops=[(0, 1)],
                semI=semI, semG=semG, semS=semS, dummy=dum)

    plsc.subcore_barrier()
    pltpu.sync_copy(acc_sp.at[sl], s0p.at[pl.ds(c * NPAD + base, SCCHUNK)])


def _idx_slots():
    return [[pltpu.VMEM((ROWS, BATCH), jnp.int32) for _ in range(6)]
            for _ in range(2)]


def _gat_slots(nops):
    return [[pltpu.VMEM((ROWS, BATCH), _f32) for _ in range(6)]
            for _ in range(nops)]


_sem_scratch = [pltpu.SemaphoreType.DMA,
                [pltpu.SemaphoreType.DMA] * 3,
                [pltpu.SemaphoreType.DMA] * 3]

_sweep1 = functools.partial(
    pl.kernel,
    out_type=(jax.ShapeDtypeStruct((NC * NPAD,), _f32),
              jax.ShapeDtypeStruct((ROWS, BATCH), _f32)),
    mesh=_mesh,
    scratch_types=[
        pltpu.VMEM_SHARED((NPAD,), _f32),
        pltpu.VMEM_SHARED((NPAD,), _f32),
        pltpu.VMEM((SCCHUNK,), _f32),
        _idx_slots(),
        _gat_slots(1),
        *_sem_scratch,
    ],
)(_sweep1_body)


def _sweep2_body(x0, s0p, ei, a1p, x1_out, dum, vals_sp, acc_sp,
                 st_a, st_b, idxs, gat, semI, semG, semS):
    c = lax.axis_index("c")
    s = lax.axis_index("s")
    base = s * SCCHUNK
    sl = pl.ds(base, SCCHUNK)

    _zero_fill(st_a, SCCHUNK)
    pltpu.sync_copy(st_a, acc_sp.at[sl])

    pltpu.sync_copy(s0p.at[pl.ds(base, SCCHUNK)], st_a)
    pltpu.sync_copy(s0p.at[pl.ds(NPAD + base, SCCHUNK)], st_b)
    _ew(st_a, SCCHUNK, lambda a, b: a + b, st_a, st_b)
    pltpu.sync_copy(x0.at[sl], st_b)
    _ew(st_a, SCCHUNK, lambda a, b: b * b * b * a, st_a, st_b)

    @pl.when(c == 0)
    def _():
        pltpu.sync_copy(st_a, x1_out.at[sl])

    _ew(st_a, SCCHUNK, lambda a: a * a * a, st_a)
    pltpu.sync_copy(st_a, vals_sp.at[sl])
    plsc.subcore_barrier()

    _sweep_loop(s * NC + c, ei, vals_sp,
                idx=idxs, gats=gat, accs=[acc_sp], ops=[(0, 1)],
                semI=semI, semG=semG, semS=semS, dummy=dum)
    _sweep_loop(s * NC + c, ei, vals_sp,
                idx=idxs, gats=gat, accs=[acc_sp], ops=[(1, 0)],
                semI=semI, semG=semG, semS=semS, dummy=dum)

    plsc.subcore_barrier()
    pltpu.sync_copy(acc_sp.at[sl], a1p.at[pl.ds(c * NPAD + base, SCCHUNK)])


_sweep2 = functools.partial(
    pl.kernel,
    out_type=(
        jax.ShapeDtypeStruct((NC * NPAD,), _f32),
        jax.ShapeDtypeStruct((NPAD,), _f32),
        jax.ShapeDtypeStruct((ROWS, BATCH), _f32),
    ),
    mesh=_mesh,
    scratch_types=[
        pltpu.VMEM_SHARED((NPAD,), _f32),
        pltpu.VMEM_SHARED((NPAD,), _f32),
        pltpu.VMEM((SCCHUNK,), _f32),
        pltpu.VMEM((SCCHUNK,), _f32),
        _idx_slots(),
        _gat_slots(1),
        *_sem_scratch,
    ],
)(_sweep2_body)


def _sweep3_body(x0, x1, a1p, ei, up, g1_out, en_out, dum, vals_sp, acc_sp,
                 st_a, st_b, st_c, idxs, gat, en_v, semI, semG, semS):
    c = lax.axis_index("c")
    s = lax.axis_index("s")
    base = s * SCCHUNK
    sl = pl.ds(base, SCCHUNK)

    _zero_fill(st_a, SCCHUNK)
    pltpu.sync_copy(st_a, acc_sp.at[sl])

    pltpu.sync_copy(a1p.at[pl.ds(base, SCCHUNK)], st_a)
    pltpu.sync_copy(a1p.at[pl.ds(NPAD + base, SCCHUNK)], st_b)
    _ew(st_a, SCCHUNK, lambda a, b: a + b, st_a, st_b)
    pltpu.sync_copy(x1.at[sl], st_b)

    @pl.when(c == 0)
    def _():
        def en_body(i, acc):
            slc = pl.ds(i * L, L)
            v = st_b[slc]
            return acc + v * v * v * st_a[slc]
        en = lax.fori_loop(0, SCCHUNK // L, en_body, jnp.zeros((L,), _f32))
        en_v[...] = 0.5 * en
        pltpu.sync_copy(en_v, en_out.at[pl.ds(s * L, L)])

    _ew(st_a, SCCHUNK, lambda a, b: 3.0 * b * b * a, st_a, st_b)

    @pl.when(c == 0)
    def _():
        pltpu.sync_copy(st_a, g1_out.at[sl])

    pltpu.sync_copy(x0.at[sl], st_c)
    _ew(st_a, SCCHUNK, lambda a, x: a * x * x * x, st_a, st_c)
    pltpu.sync_copy(st_a, vals_sp.at[sl])
    plsc.subcore_barrier()

    _sweep_loop(s * NC + c, ei, vals_sp,
                idx=idxs, gats=gat, accs=[acc_sp], ops=[(1, 0)],
                semI=semI, semG=semG, semS=semS, dummy=dum)

    plsc.subcore_barrier()
    pltpu.sync_copy(acc_sp.at[sl], up.at[pl.ds(c * NPAD + base, SCCHUNK)])


_sweep3 = functools.partial(
    pl.kernel,
    out_type=(
        jax.ShapeDtypeStruct((NC * NPAD,), _f32),
        jax.ShapeDtypeStruct((NPAD,), _f32),
        jax.ShapeDtypeStruct((NS * L,), _f32),
        jax.ShapeDtypeStruct((ROWS, BATCH), _f32),
    ),
    mesh=_mesh,
    scratch_types=[
        pltpu.VMEM_SHARED((NPAD,), _f32),
        pltpu.VMEM_SHARED((NPAD,), _f32),
        pltpu.VMEM((SCCHUNK,), _f32),
        pltpu.VMEM((SCCHUNK,), _f32),
        pltpu.VMEM((SCCHUNK,), _f32),
        _idx_slots(),
        _gat_slots(1),
        pltpu.VMEM((L,), _f32),
        *_sem_scratch,
    ],
)(_sweep3_body)


def _final_body(x0, g1, s0p, up, forces, st_a, st_b, st_c):
    c = lax.axis_index("c")
    s = lax.axis_index("s")
    wid = s * NC + c
    base = wid * CHUNK
    sl = pl.ds(base, CHUNK)

    pltpu.sync_copy(up.at[pl.ds(base, CHUNK)], st_a)
    pltpu.sync_copy(up.at[pl.ds(NPAD + base, CHUNK)], st_b)
    _ew(st_a, CHUNK, lambda a, b: a + b, st_a, st_b)
    pltpu.sync_copy(s0p.at[pl.ds(base, CHUNK)], st_b)
    pltpu.sync_copy(s0p.at[pl.ds(NPAD + base, CHUNK)], st_c)
    _ew(st_b, CHUNK, lambda a, b: a + b, st_b, st_c)
    pltpu.sync_copy(g1.at[sl], st_c)
    _ew(st_a, CHUNK, lambda a, g, s0: a + g * s0, st_a, st_c, st_b)
    pltpu.sync_copy(x0.at[sl], st_b)
    _ew(st_a, CHUNK, lambda a, x: 3.0 * x * x * a, st_a, st_b)
    pltpu.sync_copy(st_a, forces.at[sl])


_final = functools.partial(
    pl.kernel,
    out_type=jax.ShapeDtypeStruct((NPAD,), _f32),
    mesh=_mesh,
    scratch_types=[
        pltpu.VMEM((CHUNK,), _f32),
        pltpu.VMEM((CHUNK,), _f32),
        pltpu.VMEM((CHUNK,), _f32),
    ],
)(_final_body)


def kernel(atomic_numbers, edge_index):
    x0 = jnp.pad(atomic_numbers, (0, NPAD - N_NODES))
    ei = edge_index.reshape(2, NBLKG, ROWS, BATCH)
    s0p, _ = _sweep1(x0, ei)
    a1p, x1, _ = _sweep2(x0, s0p, ei)
    up, g1, en, _ = _sweep3(x0, x1, a1p, ei)
    forces = _final(x0, g1, s0p, up)
    energy = jnp.sum(en).reshape(1)
    return (energy, forces[:N_NODES])
